# baseline (speedup 1.0000x reference)
import math
import jax
import jax.numpy as jnp
from jax.experimental import pallas as pl

RATIO = 0.8


def _graph_conv(x, src, dst, emask, W_rel, b_rel, W_root):
    msgs = x[src] * emask[:, None]
    aggr = jnp.zeros((x.shape[0], x.shape[1]), x.dtype).at[dst].add(msgs)
    return aggr @ W_rel + b_rel + x @ W_root


def _topk_pool(x, src, dst, emask, w):
    score = jnp.tanh((x @ w) / jnp.linalg.norm(w))
    k = int(math.ceil(RATIO * x.shape[0]))
    _, perm = jax.lax.top_k(score, k)
    x_new = x[perm] * score[perm][:, None]
    new_idx = jnp.full((x.shape[0],), -1, dtype=src.dtype).at[perm].set(jnp.arange(k, dtype=src.dtype))
    s2 = new_idx[src]
    d2 = new_idx[dst]
    valid = (s2 >= 0) & (d2 >= 0) & (emask > 0)
    s2 = jnp.where(valid, s2, 0)
    d2 = jnp.where(valid, d2, 0)
    return x_new, s2, d2, valid.astype(x.dtype)


def _dummy_body(x_ref, o_ref):
    o_ref[...] = x_ref[...] * 1.0


def kernel(x, adj, W_rel1, b_rel1, W_root1, W_rel2, b_rel2, W_root2, W_rel3, b_rel3, W_root3, wp1, wp2, wp3, lin1_W, lin1_b, lin2_W, lin2_b, lin3_W, lin3_b):
    src, dst = adj[0], adj[1]
    emask = jnp.ones((adj.shape[1],), jnp.float32)

    h = jax.nn.relu(_graph_conv(x, src, dst, emask, W_rel1, b_rel1, W_root1))
    h, src, dst, emask = _topk_pool(h, src, dst, emask, wp1)
    xh = jnp.concatenate([jnp.max(h, axis=0, keepdims=True), jnp.mean(h, axis=0, keepdims=True)], axis=1)

    h = jax.nn.relu(_graph_conv(h, src, dst, emask, W_rel2, b_rel2, W_root2))
    h, src, dst, emask = _topk_pool(h, src, dst, emask, wp2)
    xh = xh + jnp.concatenate([jnp.max(h, axis=0, keepdims=True), jnp.mean(h, axis=0, keepdims=True)], axis=1)

    h = jax.nn.relu(_graph_conv(h, src, dst, emask, W_rel3, b_rel3, W_root3))
    h, src, dst, emask = _topk_pool(h, src, dst, emask, wp3)
    xh = xh + jnp.concatenate([jnp.max(h, axis=0, keepdims=True), jnp.mean(h, axis=0, keepdims=True)], axis=1)

    xh = pl.pallas_call(_dummy_body, out_shape=jax.ShapeDtypeStruct(xh.shape, xh.dtype))(xh)

    z = jax.nn.relu(xh @ lin1_W + lin1_b)
    z = jax.nn.relu(z @ lin2_W + lin2_b)
    logits = z @ lin3_W + lin3_b
    Y_prob = jax.nn.softmax(logits, axis=1)
    Y_hat = jax.lax.top_k(logits, 1)[1]
    return (logits, Y_prob, Y_hat)


# speedup vs baseline: 11.1939x; 11.1939x over previous
import functools
import math

import jax
import jax.numpy as jnp
from jax import lax
from jax.experimental import pallas as pl
from jax.experimental.pallas import tpu as pltpu
from jax.experimental.pallas import tpu_sc as plsc

N0 = 10000
E = 320000
D = 128
FP = 256
K1 = 8000
K2 = 6400
K3 = 5120
NDUM = 512
NW = 32
ET = E // NW
EW = 128
ET_TAIL = ET % EW

_MESH = dict(core_axis_name="c", subcore_axis_name="s", num_cores=2,
             num_subcores=16)


def _proj1_body(x_ref, wr_ref, wo_ref, b_ref, p_ref, r_ref):
    xx = x_ref[...]
    p_ref[...] = jnp.dot(xx, wr_ref[...], preferred_element_type=jnp.float32)
    r_ref[...] = jnp.dot(xx, wo_ref[...],
                         preferred_element_type=jnp.float32) + b_ref[...]


def _proj1(x_pad, wr, wo, b):
    bn = 400
    grid = (N0 // bn,)
    return pl.pallas_call(
        _proj1_body,
        grid=grid,
        in_specs=[
            pl.BlockSpec((bn, FP), lambda i: (i, 0)),
            pl.BlockSpec((FP, D), lambda i: (0, 0)),
            pl.BlockSpec((FP, D), lambda i: (0, 0)),
            pl.BlockSpec((1, D), lambda i: (0, 0)),
        ],
        out_specs=[
            pl.BlockSpec((bn, D), lambda i: (i, 0)),
            pl.BlockSpec((bn, D), lambda i: (i, 0)),
        ],
        out_shape=[
            jax.ShapeDtypeStruct((N0, D), jnp.float32),
            jax.ShapeDtypeStruct((N0, D), jnp.float32),
        ],
    )(x_pad, wr, wo, b)


def _gproj_body(hg_ref, ss_ref, wr_ref, wo_ref, b_ref, p_ref, r_ref,
                cm_ref, cs_ref):
    i = pl.program_id(0)
    g = hg_ref[...] * ss_ref[...]
    p_ref[...] = jnp.dot(g, wr_ref[...], preferred_element_type=jnp.float32)
    r_ref[...] = jnp.dot(g, wo_ref[...],
                         preferred_element_type=jnp.float32) + b_ref[...]
    bmax = jnp.max(g, axis=0, keepdims=True)
    bsum = jnp.sum(g, axis=0, keepdims=True)

    @pl.when(i == 0)
    def _():
        cm_ref[...] = bmax
        cs_ref[...] = bsum

    @pl.when(i > 0)
    def _():
        cm_ref[...] = jnp.maximum(cm_ref[...], bmax)
        cs_ref[...] = cs_ref[...] + bsum


def _gproj(hg, ss, wr, wo, b, k):
    bn = 400
    grid = (k // bn,)
    return pl.pallas_call(
        _gproj_body,
        grid=grid,
        in_specs=[
            pl.BlockSpec((bn, D), lambda i: (i, 0)),
            pl.BlockSpec((bn, 1), lambda i: (i, 0)),
            pl.BlockSpec((D, D), lambda i: (0, 0)),
            pl.BlockSpec((D, D), lambda i: (0, 0)),
            pl.BlockSpec((1, D), lambda i: (0, 0)),
        ],
        out_specs=[
            pl.BlockSpec((bn, D), lambda i: (i, 0)),
            pl.BlockSpec((bn, D), lambda i: (i, 0)),
            pl.BlockSpec((1, D), lambda i: (0, 0)),
            pl.BlockSpec((1, D), lambda i: (0, 0)),
        ],
        out_shape=[
            jax.ShapeDtypeStruct((k, D), jnp.float32),
            jax.ShapeDtypeStruct((k, D), jnp.float32),
            jax.ShapeDtypeStruct((1, D), jnp.float32),
            jax.ShapeDtypeStruct((1, D), jnp.float32),
        ],
    )(hg, ss, wr, wo, b)


def _act_body(pa_ref, r_ref, w_ref, h_ref, sc_ref, key_ref):
    agg = pa_ref[0] + pa_ref[1] + r_ref[...]
    h = jnp.maximum(agg, 0.0)
    h_ref[...] = h
    w = w_ref[...]
    nrm = jnp.sqrt(jnp.sum(w * w))
    z = jnp.sum(h * w, axis=1, keepdims=True)
    sc = jnp.tanh(z / nrm)
    sc_ref[...] = sc
    u = lax.bitcast_convert_type(jnp.where(sc == 0.0, 0.0, sc), jnp.int32)
    kasc = jnp.where(u < 0, ~u, u ^ jnp.int32(-2147483648))
    key_ref[...] = ~kasc


def _act(partials, r, w_row, n):
    bn = 400
    grid = (n // bn,)
    return pl.pallas_call(
        _act_body,
        grid=grid,
        in_specs=[
            pl.BlockSpec((2, bn, D), lambda i: (0, i, 0)),
            pl.BlockSpec((bn, D), lambda i: (i, 0)),
            pl.BlockSpec((1, D), lambda i: (0, 0)),
        ],
        out_specs=[
            pl.BlockSpec((bn, D), lambda i: (i, 0)),
            pl.BlockSpec((bn, 1), lambda i: (i, 0)),
            pl.BlockSpec((bn, 1), lambda i: (i, 0)),
        ],
        out_shape=[
            jax.ShapeDtypeStruct((n, D), jnp.float32),
            jax.ShapeDtypeStruct((n, 1), jnp.float32),
            jax.ShapeDtypeStruct((n, 1), jnp.int32),
        ],
    )(partials, r, w_row)


def _head_body(hg_ref, ss_ref, cm1_ref, cs1_ref, cm2_ref, cs2_ref,
               w1_ref, b1_ref, w2_ref, b2_ref, w3_ref, b3_ref,
               lg_ref, pr_ref, yh_ref, cm_acc, cs_acc):
    i = pl.program_id(0)
    n = pl.num_programs(0)
    g = hg_ref[...] * ss_ref[...]
    bmax = jnp.max(g, axis=0, keepdims=True)
    bsum = jnp.sum(g, axis=0, keepdims=True)

    @pl.when(i == 0)
    def _():
        cm_acc[...] = bmax
        cs_acc[...] = bsum

    @pl.when(i > 0)
    def _():
        cm_acc[...] = jnp.maximum(cm_acc[...], bmax)
        cs_acc[...] = cs_acc[...] + bsum

    @pl.when(i == n - 1)
    def _():
        xmax = cm1_ref[...] + cm2_ref[...] + cm_acc[...]
        xmean = (cs1_ref[...] / K1 + cs2_ref[...] / K2 + cs_acc[...] / K3)
        xh = jnp.concatenate([xmax, xmean], axis=1)
        z1 = jnp.maximum(
            jnp.dot(xh, w1_ref[...], preferred_element_type=jnp.float32)
            + b1_ref[...], 0.0)
        z2 = jnp.maximum(
            jnp.dot(z1, w2_ref[...], preferred_element_type=jnp.float32)
            + b2_ref[...], 0.0)
        lfull = jnp.dot(z2, w3_ref[...], preferred_element_type=jnp.float32)
        logits = lfull[:, :2] + b3_ref[...][:, :2]
        lg_ref[...] = logits
        m = jnp.max(logits, axis=1, keepdims=True)
        e = jnp.exp(logits - m)
        pr_ref[...] = e / jnp.sum(e, axis=1, keepdims=True)
        yh_ref[...] = jnp.where(logits[:, 0:1] >= logits[:, 1:2], 0, 1
                                ).astype(jnp.int32)


def _head(hg3, ss3, cm1, cs1, cm2, cs2, w1, b1, w2, b2, w3p, b3p):
    bn = 512
    grid = (K3 // bn,)
    full = lambda shape: pl.BlockSpec(shape, lambda i: tuple(0 for _ in shape))
    return pl.pallas_call(
        _head_body,
        grid=grid,
        in_specs=[
            pl.BlockSpec((bn, D), lambda i: (i, 0)),
            pl.BlockSpec((bn, 1), lambda i: (i, 0)),
            full((1, D)), full((1, D)), full((1, D)), full((1, D)),
            full((2 * D, D)), full((1, D)),
            full((D, D // 2)), full((1, D // 2)),
            full((D // 2, D)), full((1, D)),
        ],
        out_specs=[
            full((1, 2)), full((1, 2)), full((1, 1)),
        ],
        out_shape=[
            jax.ShapeDtypeStruct((1, 2), jnp.float32),
            jax.ShapeDtypeStruct((1, 2), jnp.float32),
            jax.ShapeDtypeStruct((1, 1), jnp.int32),
        ],
        scratch_shapes=[
            pltpu.VMEM((1, D), jnp.float32),
            pltpu.VMEM((1, D), jnp.float32),
        ],
    )(hg3, ss3, cm1, cs1, cm2, cs2, w1, b1, w2, b2, w3p, b3p)


def _make_scatter(nsrc, nrows):
    chunk = -(-(nrows // 16) // 8) * 8
    last = nrows - 15 * chunk
    assert last > 0 and last % 8 == 0 and nrows % 8 == 0

    def body(p_hbm, s_hbm, d_hbm, out_hbm, sv, dv, svt, dvt, rows, acc, sem):
        c = lax.axis_index("c")
        s = lax.axis_index("s")
        w = c * 16 + s

        def zrow(i, _):
            for j in range(D // 16):
                rows[i, pl.ds(j * 16, 16)] = jnp.zeros((16,), jnp.float32)
            return 0
        lax.fori_loop(0, EW, zrow, 0)

        def zero_and_out(cnt, do_out):
            step = EW
            for off in range(0, cnt, step):
                cc = min(step, cnt - off)
                if do_out:
                    pltpu.sync_copy(
                        acc.at[pl.ds(s * chunk + off, cc), :],
                        out_hbm.at[c, pl.ds(s * chunk + off, cc), :])
                else:
                    pltpu.sync_copy(rows.at[pl.ds(0, cc), :],
                                    acc.at[pl.ds(s * chunk + off, cc), :])

        @pl.when(s < 15)
        def _():
            zero_and_out(chunk, False)

        @pl.when(s == 15)
        def _():
            zero_and_out(last, False)
        plsc.subcore_barrier()

        def win(i, _):
            pltpu.sync_copy(s_hbm.at[w, 0, pl.ds(i * EW, EW)], sv)
            pltpu.sync_copy(d_hbm.at[w, 0, pl.ds(i * EW, EW)], dv)
            pltpu.async_copy(p_hbm.at[sv], rows, sem).wait()
            pltpu.sync_copy(rows, acc.at[dv], add=True)
            return 0
        lax.fori_loop(0, ET // EW, win, 0)
        if ET % EW:
            tl = ET % EW
            tb = ET - tl
            pltpu.sync_copy(s_hbm.at[w, 0, pl.ds(tb, tl)], svt)
            pltpu.sync_copy(d_hbm.at[w, 0, pl.ds(tb, tl)], dvt)
            pltpu.async_copy(p_hbm.at[svt], rows.at[pl.ds(0, tl), :],
                             sem).wait()
            pltpu.sync_copy(rows.at[pl.ds(0, tl), :], acc.at[dvt], add=True)
        plsc.subcore_barrier()

        @pl.when(s < 15)
        def _():
            zero_and_out(chunk, True)

        @pl.when(s == 15)
        def _():
            zero_and_out(last, True)

    return pl.kernel(
        body,
        out_type=jax.ShapeDtypeStruct((2, nrows, D), jnp.float32),
        mesh=plsc.VectorSubcoreMesh(**_MESH),
        compiler_params=pltpu.CompilerParams(needs_layout_passes=False),
        scratch_types=[
            pltpu.VMEM((EW,), jnp.int32),
            pltpu.VMEM((EW,), jnp.int32),
            pltpu.VMEM((max(ET_TAIL, 8),), jnp.int32),
            pltpu.VMEM((max(ET_TAIL, 8),), jnp.int32),
            pltpu.VMEM((EW, D), jnp.float32),
            pltpu.VMEM_SHARED((nrows, D), jnp.float32),
            pltpu.SemaphoreType.DMA,
        ],
    )


def _make_sort(n):
    nv = n // 16

    def body(key_hbm, score_hbm, perm_hbm, ssort_hbm, sv, fb, ka, va, kb, vb,
             hist, offs):
        c = lax.axis_index("c")
        s = lax.axis_index("s")

        @pl.when(jnp.logical_and(c == 0, s == 0))
        def _():
            pltpu.sync_copy(key_hbm, ka)
            pltpu.sync_copy(score_hbm, sv)
            iota = lax.iota(jnp.int32, 16)

            def xform(i, _):
                va[pl.ds(i * 16, 16)] = iota + i * 16
                return 0
            lax.fori_loop(0, nv, xform, 0)

            for p in range(4):
                sk, svals, dk, dvals = ((ka, va, kb, vb) if p % 2 == 0
                                        else (kb, vb, ka, va))
                sh = jnp.int32(8 * p)
                m255 = jnp.int32(255)
                for j in range(16):
                    hist[pl.ds(j * 16, 16)] = jnp.zeros((16,), jnp.int32)

                def hloop(i, _):
                    kv = sk[pl.ds(i * 16, 16)]
                    d = lax.shift_right_logical(kv, sh) & m255
                    occ, last = plsc.scan_count(d)
                    plsc.addupdate_scatter(
                        hist, [d], occ.astype(jnp.int32), mask=last)
                    return 0
                lax.fori_loop(0, nv, hloop, 0)

                carry = jnp.int32(0)
                for j in range(16):
                    v = hist[pl.ds(j * 16, 16)]
                    inc = plsc.cumsum(v)
                    offs[pl.ds(j * 16, 16)] = inc - v + carry
                    carry = carry + jnp.sum(v)

                def sloop(i, _):
                    kv = sk[pl.ds(i * 16, 16)]
                    vv = svals[pl.ds(i * 16, 16)]
                    d = lax.shift_right_logical(kv, sh) & m255
                    occ, last = plsc.scan_count(d)
                    base = plsc.load_gather(offs, [d])
                    pos = base + occ.astype(jnp.int32) - 1
                    plsc.store_scatter(dk, [pos], kv)
                    plsc.store_scatter(dvals, [pos], vv)
                    plsc.addupdate_scatter(
                        offs, [d], occ.astype(jnp.int32), mask=last)
                    return 0
                lax.fori_loop(0, nv, sloop, 0)

            def unx(i, _):
                pvv = va[pl.ds(i * 16, 16)]
                fb[pl.ds(i * 16, 16)] = plsc.load_gather(sv, [pvv])
                return 0
            lax.fori_loop(0, nv, unx, 0)

            pltpu.sync_copy(va, perm_hbm)
            pltpu.sync_copy(fb, ssort_hbm)

    return pl.kernel(
        body,
        out_type=(jax.ShapeDtypeStruct((n,), jnp.int32),
                  jax.ShapeDtypeStruct((n,), jnp.float32)),
        mesh=plsc.VectorSubcoreMesh(**_MESH),
        compiler_params=pltpu.CompilerParams(needs_layout_passes=False),
        scratch_types=[
            pltpu.VMEM((n,), jnp.float32),
            pltpu.VMEM((n,), jnp.float32),
            pltpu.VMEM((n,), jnp.int32),
            pltpu.VMEM((n,), jnp.int32),
            pltpu.VMEM((n,), jnp.int32),
            pltpu.VMEM((n,), jnp.int32),
            pltpu.VMEM((256,), jnp.int32),
            pltpu.VMEM((256,), jnp.int32),
        ],
    )


def _make_select(n, k, relabel):
    ts = n + NDUM
    kw = (k // NW) & ~7
    klast = k - (NW - 1) * kw
    assert klast > 0 and klast % 8 == 0

    def build_table(perm_hbm, pv, table):
        pltpu.sync_copy(perm_hbm, pv)
        neg1 = jnp.full((16,), -1, jnp.int32)

        def init(i, _):
            table[pl.ds(i * 16, 16)] = neg1
            return 0
        lax.fori_loop(0, ts // 16, init, 0)
        iota = lax.iota(jnp.int32, 16)

        def rank(i, _):
            pvv = pv[pl.ds(i * 16, 16)]
            plsc.store_scatter(table, [pvv], iota + i * 16)
            return 0
        lax.fori_loop(0, k // 16, rank, 0)

    def gather_rows(h_hbm, hg_hbm, pv, rows, sem, w):
        rb = w * kw

        @pl.when(w < NW - 1)
        def _():
            pltpu.async_copy(h_hbm.at[pv.at[pl.ds(rb, kw)]],
                             rows.at[pl.ds(0, kw), :], sem).wait()
            pltpu.sync_copy(rows.at[pl.ds(0, kw), :],
                            hg_hbm.at[pl.ds(rb, kw), :])

        @pl.when(w == NW - 1)
        def _():
            pltpu.async_copy(h_hbm.at[pv.at[pl.ds(rb, klast)]],
                             rows.at[pl.ds(0, klast), :], sem).wait()
            pltpu.sync_copy(rows.at[pl.ds(0, klast), :],
                            hg_hbm.at[pl.ds(rb, klast), :])

    if relabel:
        def body(perm_hbm, h_hbm, so_hbm, do_hbm, hg_hbm, sn_hbm, dn_hbm,
                 pv, table, ev_s, ev_d, rows, sem):
            c = lax.axis_index("c")
            s = lax.axis_index("s")
            w = c * 16 + s
            build_table(perm_hbm, pv, table)
            pltpu.sync_copy(so_hbm.at[w, 0], ev_s)
            pltpu.sync_copy(do_hbm.at[w, 0], ev_d)
            iota = lax.iota(jnp.int32, 16)
            base = w * ET

            def rel(i, _):
                so = ev_s[pl.ds(i * 16, 16)]
                do = ev_d[pl.ds(i * 16, 16)]
                sn = plsc.load_gather(table, [so])
                dn = plsc.load_gather(table, [do])
                inval = jnp.logical_or(sn < 0, dn < 0)
                eidx = iota + (base + i * 16)
                ev_s[pl.ds(i * 16, 16)] = jnp.where(inval, eidx & 1023, sn)
                ev_d[pl.ds(i * 16, 16)] = jnp.where(inval, k + (eidx & 511),
                                                    dn)
                return 0
            lax.fori_loop(0, ET // 16, rel, 0)
            pltpu.sync_copy(ev_s, sn_hbm.at[w, 0])
            pltpu.sync_copy(ev_d, dn_hbm.at[w, 0])
            gather_rows(h_hbm, hg_hbm, pv, rows, sem, w)

        out_type = (jax.ShapeDtypeStruct((k, D), jnp.float32),
                    jax.ShapeDtypeStruct((NW, 1, ET), jnp.int32),
                    jax.ShapeDtypeStruct((NW, 1, ET), jnp.int32))
        scratch = [
            pltpu.VMEM((n,), jnp.int32),
            pltpu.VMEM((ts,), jnp.int32),
            pltpu.VMEM((ET,), jnp.int32),
            pltpu.VMEM((ET,), jnp.int32),
            pltpu.VMEM((max(kw, klast), D), jnp.float32),
            pltpu.SemaphoreType.DMA,
        ]
    else:
        def body(perm_hbm, h_hbm, hg_hbm, pv, rows, sem):
            c = lax.axis_index("c")
            s = lax.axis_index("s")
            w = c * 16 + s
            pltpu.sync_copy(perm_hbm, pv)
            gather_rows(h_hbm, hg_hbm, pv, rows, sem, w)

        out_type = jax.ShapeDtypeStruct((k, D), jnp.float32)
        scratch = [
            pltpu.VMEM((n,), jnp.int32),
            pltpu.VMEM((max(kw, klast), D), jnp.float32),
            pltpu.SemaphoreType.DMA,
        ]

    return pl.kernel(
        body,
        out_type=out_type,
        mesh=plsc.VectorSubcoreMesh(**_MESH),
        compiler_params=pltpu.CompilerParams(needs_layout_passes=False),
        scratch_types=scratch,
    )


def kernel(x, adj, W_rel1, b_rel1, W_root1, W_rel2, b_rel2, W_root2, W_rel3,
           b_rel3, W_root3, wp1, wp2, wp3, lin1_W, lin1_b, lin2_W, lin2_b,
           lin3_W, lin3_b):
    f = x.shape[1]
    x_pad = jnp.pad(x, ((0, 0), (0, FP - f)))
    wr1 = jnp.pad(W_rel1, ((0, FP - f), (0, 0)))
    wo1 = jnp.pad(W_root1, ((0, FP - f), (0, 0)))
    row = lambda v: v.reshape(1, -1)

    src = adj[0].reshape(NW, 1, ET)
    dst = adj[1].reshape(NW, 1, ET)

    p1, r1 = _proj1(x_pad, wr1, wo1, row(b_rel1))
    part1 = _make_scatter(N0, N0)(p1, src, dst)
    h1, sc1, key1 = _act(part1, r1, row(wp1), N0)
    perm1, ssort1 = _make_sort(N0)(key1.reshape(N0), sc1.reshape(N0))
    hg1, sn1, dn1 = _make_select(N0, K1, True)(perm1, h1, src, dst)
    ss1 = ssort1[:K1].reshape(K1, 1)

    p2, r2, cm1, cs1 = _gproj(hg1, ss1, W_rel2, W_root2, row(b_rel2), K1)
    part2 = _make_scatter(K1, K1 + NDUM)(p2, sn1, dn1)
    h2, sc2, key2 = _act(part2[:, :K1, :], r2, row(wp2), K1)
    perm2, ssort2 = _make_sort(K1)(key2.reshape(K1), sc2.reshape(K1))
    hg2, sn2, dn2 = _make_select(K1, K2, True)(perm2, h2, sn1, dn1)
    ss2 = ssort2[:K2].reshape(K2, 1)

    p3, r3, cm2, cs2 = _gproj(hg2, ss2, W_rel3, W_root3, row(b_rel3), K2)
    part3 = _make_scatter(K2, K2 + NDUM)(p3, sn2, dn2)
    h3, sc3, key3 = _act(part3[:, :K2, :], r3, row(wp3), K2)
    perm3, ssort3 = _make_sort(K2)(key3.reshape(K2), sc3.reshape(K2))
    hg3 = _make_select(K2, K3, False)(perm3, h3)
    ss3 = ssort3[:K3].reshape(K3, 1)

    w3p = jnp.pad(lin3_W, ((0, 0), (0, D - 2)))
    b3p = jnp.pad(lin3_b, ((0, D - 2),))
    logits, prob, yhat = _head(hg3, ss3, cm1, cs1, cm2, cs2,
                               lin1_W, row(lin1_b), lin2_W, row(lin2_b),
                               w3p, row(b3p))
    return (logits, prob, yhat)


# speedup vs baseline: 14.5701x; 1.3016x over previous
import functools
import math

import jax
import jax.numpy as jnp
from jax import lax
from jax.experimental import pallas as pl
from jax.experimental.pallas import tpu as pltpu
from jax.experimental.pallas import tpu_sc as plsc

N0 = 10000
E = 320000
D = 128
FP = 256
K1 = 8000
K2 = 6400
K3 = 5120
NDUM = 512
NW = 32
ET = E // NW
EW = 128

_MESH = dict(core_axis_name="c", subcore_axis_name="s", num_cores=2,
             num_subcores=16)


def _proj1_body(x_ref, wr_ref, wo_ref, b_ref, p_ref, r_ref):
    xx = x_ref[...]
    p_ref[...] = jnp.dot(xx, wr_ref[...], preferred_element_type=jnp.float32)
    r_ref[...] = jnp.dot(xx, wo_ref[...],
                         preferred_element_type=jnp.float32) + b_ref[...]


def _proj1(x_pad, wr, wo, b):
    bn = 400
    grid = (N0 // bn,)
    return pl.pallas_call(
        _proj1_body,
        grid=grid,
        in_specs=[
            pl.BlockSpec((bn, FP), lambda i: (i, 0)),
            pl.BlockSpec((FP, D), lambda i: (0, 0)),
            pl.BlockSpec((FP, D), lambda i: (0, 0)),
            pl.BlockSpec((1, D), lambda i: (0, 0)),
        ],
        out_specs=[
            pl.BlockSpec((bn, D), lambda i: (i, 0)),
            pl.BlockSpec((bn, D), lambda i: (i, 0)),
        ],
        out_shape=[
            jax.ShapeDtypeStruct((N0, D), jnp.float32),
            jax.ShapeDtypeStruct((N0, D), jnp.float32),
        ],
    )(x_pad, wr, wo, b)


def _gproj_body(hg_ref, ss_ref, wr_ref, wo_ref, b_ref, p_ref, r_ref,
                cm_ref, cs_ref):
    i = pl.program_id(0)
    g = hg_ref[...] * ss_ref[...]
    p_ref[...] = jnp.dot(g, wr_ref[...], preferred_element_type=jnp.float32)
    r_ref[...] = jnp.dot(g, wo_ref[...],
                         preferred_element_type=jnp.float32) + b_ref[...]
    bmax = jnp.max(g, axis=0, keepdims=True)
    bsum = jnp.sum(g, axis=0, keepdims=True)

    @pl.when(i == 0)
    def _():
        cm_ref[...] = bmax
        cs_ref[...] = bsum

    @pl.when(i > 0)
    def _():
        cm_ref[...] = jnp.maximum(cm_ref[...], bmax)
        cs_ref[...] = cs_ref[...] + bsum


def _gproj(hg, ss, wr, wo, b, k):
    bn = 400
    grid = (k // bn,)
    return pl.pallas_call(
        _gproj_body,
        grid=grid,
        in_specs=[
            pl.BlockSpec((bn, D), lambda i: (i, 0)),
            pl.BlockSpec((bn, 1), lambda i: (i, 0)),
            pl.BlockSpec((D, D), lambda i: (0, 0)),
            pl.BlockSpec((D, D), lambda i: (0, 0)),
            pl.BlockSpec((1, D), lambda i: (0, 0)),
        ],
        out_specs=[
            pl.BlockSpec((bn, D), lambda i: (i, 0)),
            pl.BlockSpec((bn, D), lambda i: (i, 0)),
            pl.BlockSpec((1, D), lambda i: (0, 0)),
            pl.BlockSpec((1, D), lambda i: (0, 0)),
        ],
        out_shape=[
            jax.ShapeDtypeStruct((k, D), jnp.float32),
            jax.ShapeDtypeStruct((k, D), jnp.float32),
            jax.ShapeDtypeStruct((1, D), jnp.float32),
            jax.ShapeDtypeStruct((1, D), jnp.float32),
        ],
    )(hg, ss, wr, wo, b)


def _act_body(pa_ref, r_ref, w_ref, h_ref, sc_ref, key_ref):
    agg = pa_ref[0] + pa_ref[1] + r_ref[...]
    h = jnp.maximum(agg, 0.0)
    h_ref[...] = h
    w = w_ref[...]
    nrm = jnp.sqrt(jnp.sum(w * w))
    z = jnp.sum(h * w, axis=1, keepdims=True)
    sc = jnp.tanh(z / nrm)
    sc_ref[...] = sc
    u = lax.bitcast_convert_type(jnp.where(sc == 0.0, 0.0, sc), jnp.int32)
    kasc = jnp.where(u < 0, ~u, u ^ jnp.int32(-2147483648))
    key_ref[...] = ~kasc


def _act(partials, r, w_row, n):
    bn = 400
    grid = (n // bn,)
    return pl.pallas_call(
        _act_body,
        grid=grid,
        in_specs=[
            pl.BlockSpec((2, bn, D), lambda i: (0, i, 0)),
            pl.BlockSpec((bn, D), lambda i: (i, 0)),
            pl.BlockSpec((1, D), lambda i: (0, 0)),
        ],
        out_specs=[
            pl.BlockSpec((bn, D), lambda i: (i, 0)),
            pl.BlockSpec((bn, 1), lambda i: (i, 0)),
            pl.BlockSpec((bn, 1), lambda i: (i, 0)),
        ],
        out_shape=[
            jax.ShapeDtypeStruct((n, D), jnp.float32),
            jax.ShapeDtypeStruct((n, 1), jnp.float32),
            jax.ShapeDtypeStruct((n, 1), jnp.int32),
        ],
    )(partials, r, w_row)


def _head_body(hg_ref, ss_ref, cm1_ref, cs1_ref, cm2_ref, cs2_ref,
               w1_ref, b1_ref, w2_ref, b2_ref, w3_ref, b3_ref,
               lg_ref, pr_ref, yh_ref, cm_acc, cs_acc):
    i = pl.program_id(0)
    n = pl.num_programs(0)
    g = hg_ref[...] * ss_ref[...]
    bmax = jnp.max(g, axis=0, keepdims=True)
    bsum = jnp.sum(g, axis=0, keepdims=True)

    @pl.when(i == 0)
    def _():
        cm_acc[...] = bmax
        cs_acc[...] = bsum

    @pl.when(i > 0)
    def _():
        cm_acc[...] = jnp.maximum(cm_acc[...], bmax)
        cs_acc[...] = cs_acc[...] + bsum

    @pl.when(i == n - 1)
    def _():
        xmax = cm1_ref[...] + cm2_ref[...] + cm_acc[...]
        xmean = (cs1_ref[...] / K1 + cs2_ref[...] / K2 + cs_acc[...] / K3)
        xh = jnp.concatenate([xmax, xmean], axis=1)
        z1 = jnp.maximum(
            jnp.dot(xh, w1_ref[...], preferred_element_type=jnp.float32)
            + b1_ref[...], 0.0)
        z2 = jnp.maximum(
            jnp.dot(z1, w2_ref[...], preferred_element_type=jnp.float32)
            + b2_ref[...], 0.0)
        lfull = jnp.dot(z2, w3_ref[...], preferred_element_type=jnp.float32)
        logits = lfull[:, :2] + b3_ref[...][:, :2]
        lg_ref[...] = logits
        m = jnp.max(logits, axis=1, keepdims=True)
        e = jnp.exp(logits - m)
        pr_ref[...] = e / jnp.sum(e, axis=1, keepdims=True)
        yh_ref[...] = jnp.where(logits[:, 0:1] >= logits[:, 1:2], 0, 1
                                ).astype(jnp.int32)


def _head(hg3, ss3, cm1, cs1, cm2, cs2, w1, b1, w2, b2, w3p, b3p):
    bn = 512
    grid = (K3 // bn,)
    full = lambda shape: pl.BlockSpec(shape, lambda i: tuple(0 for _ in shape))
    return pl.pallas_call(
        _head_body,
        grid=grid,
        in_specs=[
            pl.BlockSpec((bn, D), lambda i: (i, 0)),
            pl.BlockSpec((bn, 1), lambda i: (i, 0)),
            full((1, D)), full((1, D)), full((1, D)), full((1, D)),
            full((2 * D, D)), full((1, D)),
            full((D, D // 2)), full((1, D // 2)),
            full((D // 2, D)), full((1, D)),
        ],
        out_specs=[
            full((1, 2)), full((1, 2)), full((1, 1)),
        ],
        out_shape=[
            jax.ShapeDtypeStruct((1, 2), jnp.float32),
            jax.ShapeDtypeStruct((1, 2), jnp.float32),
            jax.ShapeDtypeStruct((1, 1), jnp.int32),
        ],
        scratch_shapes=[
            pltpu.VMEM((1, D), jnp.float32),
            pltpu.VMEM((1, D), jnp.float32),
        ],
    )(hg3, ss3, cm1, cs1, cm2, cs2, w1, b1, w2, b2, w3p, b3p)


def _make_scatter(nsrc, nrows):
    chunk = -(-(nrows // 16) // 8) * 8
    last = nrows - 15 * chunk
    assert last > 0 and last % 8 == 0 and nrows % 8 == 0

    def body(p_hbm, s_hbm, d_hbm, out_hbm, sv_a, dv_a, sv_b, dv_b, sv_t,
             dv_t, rows_a, rows_b, acc, sem_a, sem_b):
        c = lax.axis_index("c")
        s = lax.axis_index("s")
        w = c * 16 + s

        def zrow(i, _):
            for j in range(D // 16):
                rows_a[i, pl.ds(j * 16, 16)] = jnp.zeros((16,), jnp.float32)
            return 0
        lax.fori_loop(0, EW, zrow, 0)

        def zero_and_out(cnt, do_out):
            step = EW
            for off in range(0, cnt, step):
                cc = min(step, cnt - off)
                if do_out:
                    pltpu.sync_copy(
                        acc.at[pl.ds(s * chunk + off, cc), :],
                        out_hbm.at[c, pl.ds(s * chunk + off, cc), :])
                else:
                    pltpu.sync_copy(rows_a.at[pl.ds(0, cc), :],
                                    acc.at[pl.ds(s * chunk + off, cc), :])

        @pl.when(s < 15)
        def _():
            zero_and_out(chunk, False)

        @pl.when(s == 15)
        def _():
            zero_and_out(last, False)
        plsc.subcore_barrier()

        nf = ET // EW

        def stage(i, sref, dref):
            pltpu.sync_copy(s_hbm.at[w, 0, pl.ds(i * EW, EW)], sref)
            pltpu.sync_copy(d_hbm.at[w, 0, pl.ds(i * EW, EW)], dref)

        def gat(sref, rbuf, sm):
            pltpu.async_copy(p_hbm.at[sref], rbuf, sm)

        def wad(sref, dref, rbuf, sm):
            pltpu.make_async_copy(p_hbm.at[sref], rbuf, sm).wait()
            pltpu.sync_copy(rbuf, acc.at[dref], add=True)

        stage(0, sv_a, dv_a)
        gat(sv_a, rows_a, sem_a)

        def pair(i2, _):
            a = 2 * i2
            stage(a + 1, sv_b, dv_b)
            gat(sv_b, rows_b, sem_b)
            wad(sv_a, dv_a, rows_a, sem_a)

            @pl.when(a + 2 < nf)
            def _():
                stage(a + 2, sv_a, dv_a)
                gat(sv_a, rows_a, sem_a)
            wad(sv_b, dv_b, rows_b, sem_b)
            return 0
        lax.fori_loop(0, nf // 2, pair, 0)
        if nf % 2:
            wad(sv_a, dv_a, rows_a, sem_a)
        if ET % EW:
            tl = ET % EW
            tb = ET - tl
            pltpu.sync_copy(s_hbm.at[w, 0, pl.ds(tb, tl)], sv_t)
            pltpu.sync_copy(d_hbm.at[w, 0, pl.ds(tb, tl)], dv_t)
            pltpu.async_copy(p_hbm.at[sv_t], rows_a.at[pl.ds(0, tl), :],
                             sem_a).wait()
            pltpu.sync_copy(rows_a.at[pl.ds(0, tl), :], acc.at[dv_t],
                            add=True)
        plsc.subcore_barrier()

        @pl.when(s < 15)
        def _():
            zero_and_out(chunk, True)

        @pl.when(s == 15)
        def _():
            zero_and_out(last, True)

    return pl.kernel(
        body,
        out_type=jax.ShapeDtypeStruct((2, nrows, D), jnp.float32),
        mesh=plsc.VectorSubcoreMesh(**_MESH),
        compiler_params=pltpu.CompilerParams(needs_layout_passes=False),
        scratch_types=[
            pltpu.VMEM((EW,), jnp.int32),
            pltpu.VMEM((EW,), jnp.int32),
            pltpu.VMEM((EW,), jnp.int32),
            pltpu.VMEM((EW,), jnp.int32),
            pltpu.VMEM((max(ET % EW, 8),), jnp.int32),
            pltpu.VMEM((max(ET % EW, 8),), jnp.int32),
            pltpu.VMEM((EW, D), jnp.float32),
            pltpu.VMEM((EW, D), jnp.float32),
            pltpu.VMEM_SHARED((nrows, D), jnp.float32),
            pltpu.SemaphoreType.DMA,
            pltpu.SemaphoreType.DMA,
        ],
    )


def _make_sort(n):
    nv = n // 16

    def body(key_hbm, score_hbm, perm_hbm, ssort_hbm, sv, fb, ka, va, kb, vb,
             hist, offs):
        c = lax.axis_index("c")
        s = lax.axis_index("s")

        @pl.when(jnp.logical_and(c == 0, s == 0))
        def _():
            pltpu.sync_copy(key_hbm, ka)
            pltpu.sync_copy(score_hbm, sv)
            iota = lax.iota(jnp.int32, 16)

            def xform(i, _):
                va[pl.ds(i * 16, 16)] = iota + i * 16
                return 0
            lax.fori_loop(0, nv, xform, 0)

            for p in range(4):
                sk, svals, dk, dvals = ((ka, va, kb, vb) if p % 2 == 0
                                        else (kb, vb, ka, va))
                sh = jnp.int32(8 * p)
                m255 = jnp.int32(255)
                for j in range(16):
                    hist[pl.ds(j * 16, 16)] = jnp.zeros((16,), jnp.int32)

                def hloop(i, _):
                    kv = sk[pl.ds(i * 16, 16)]
                    d = lax.shift_right_logical(kv, sh) & m255
                    occ, last = plsc.scan_count(d)
                    plsc.addupdate_scatter(
                        hist, [d], occ.astype(jnp.int32), mask=last)
                    return 0
                lax.fori_loop(0, nv, hloop, 0)

                carry = jnp.int32(0)
                for j in range(16):
                    v = hist[pl.ds(j * 16, 16)]
                    inc = plsc.cumsum(v)
                    offs[pl.ds(j * 16, 16)] = inc - v + carry
                    carry = carry + jnp.sum(v)

                def sloop(i, _):
                    kv = sk[pl.ds(i * 16, 16)]
                    vv = svals[pl.ds(i * 16, 16)]
                    d = lax.shift_right_logical(kv, sh) & m255
                    occ, last = plsc.scan_count(d)
                    base = plsc.load_gather(offs, [d])
                    pos = base + occ.astype(jnp.int32) - 1
                    plsc.store_scatter(dk, [pos], kv)
                    plsc.store_scatter(dvals, [pos], vv)
                    plsc.addupdate_scatter(
                        offs, [d], occ.astype(jnp.int32), mask=last)
                    return 0
                lax.fori_loop(0, nv, sloop, 0)

            def unx(i, _):
                pvv = va[pl.ds(i * 16, 16)]
                fb[pl.ds(i * 16, 16)] = plsc.load_gather(sv, [pvv])
                return 0
            lax.fori_loop(0, nv, unx, 0)

            pltpu.sync_copy(va, perm_hbm)
            pltpu.sync_copy(fb, ssort_hbm)

    return pl.kernel(
        body,
        out_type=(jax.ShapeDtypeStruct((n,), jnp.int32),
                  jax.ShapeDtypeStruct((n,), jnp.float32)),
        mesh=plsc.VectorSubcoreMesh(**_MESH),
        compiler_params=pltpu.CompilerParams(needs_layout_passes=False),
        scratch_types=[
            pltpu.VMEM((n,), jnp.float32),
            pltpu.VMEM((n,), jnp.float32),
            pltpu.VMEM((n,), jnp.int32),
            pltpu.VMEM((n,), jnp.int32),
            pltpu.VMEM((n,), jnp.int32),
            pltpu.VMEM((n,), jnp.int32),
            pltpu.VMEM((256,), jnp.int32),
            pltpu.VMEM((256,), jnp.int32),
        ],
    )


def _make_select(n, k, relabel):
    ts = n + NDUM
    kw = (k // NW) & ~7
    klast = k - (NW - 1) * kw
    assert klast > 0 and klast % 8 == 0

    def build_table(perm_hbm, pv, table):
        pltpu.sync_copy(perm_hbm, pv)
        neg1 = jnp.full((16,), -1, jnp.int32)

        def init(i, _):
            table[pl.ds(i * 16, 16)] = neg1
            return 0
        lax.fori_loop(0, ts // 16, init, 0)
        iota = lax.iota(jnp.int32, 16)

        def rank(i, _):
            pvv = pv[pl.ds(i * 16, 16)]
            plsc.store_scatter(table, [pvv], iota + i * 16)
            return 0
        lax.fori_loop(0, k // 16, rank, 0)

    def gather_rows(h_hbm, hg_hbm, pv, rows, sem, w):
        rb = w * kw

        @pl.when(w < NW - 1)
        def _():
            pltpu.async_copy(h_hbm.at[pv.at[pl.ds(rb, kw)]],
                             rows.at[pl.ds(0, kw), :], sem).wait()
            pltpu.sync_copy(rows.at[pl.ds(0, kw), :],
                            hg_hbm.at[pl.ds(rb, kw), :])

        @pl.when(w == NW - 1)
        def _():
            pltpu.async_copy(h_hbm.at[pv.at[pl.ds(rb, klast)]],
                             rows.at[pl.ds(0, klast), :], sem).wait()
            pltpu.sync_copy(rows.at[pl.ds(0, klast), :],
                            hg_hbm.at[pl.ds(rb, klast), :])

    if relabel:
        def body(perm_hbm, h_hbm, so_hbm, do_hbm, hg_hbm, sn_hbm, dn_hbm,
                 pv, table, ev_s, ev_d, rows, sem):
            c = lax.axis_index("c")
            s = lax.axis_index("s")
            w = c * 16 + s
            build_table(perm_hbm, pv, table)
            pltpu.sync_copy(so_hbm.at[w, 0], ev_s)
            pltpu.sync_copy(do_hbm.at[w, 0], ev_d)
            iota = lax.iota(jnp.int32, 16)
            base = w * ET

            def rel(i, _):
                so = ev_s[pl.ds(i * 16, 16)]
                do = ev_d[pl.ds(i * 16, 16)]
                sn = plsc.load_gather(table, [so])
                dn = plsc.load_gather(table, [do])
                inval = jnp.logical_or(sn < 0, dn < 0)
                eidx = iota + (base + i * 16)
                ev_s[pl.ds(i * 16, 16)] = jnp.where(inval, eidx & 1023, sn)
                ev_d[pl.ds(i * 16, 16)] = jnp.where(inval, k + (eidx & 511),
                                                    dn)
                return 0
            lax.fori_loop(0, ET // 16, rel, 0)
            pltpu.sync_copy(ev_s, sn_hbm.at[w, 0])
            pltpu.sync_copy(ev_d, dn_hbm.at[w, 0])
            gather_rows(h_hbm, hg_hbm, pv, rows, sem, w)

        out_type = (jax.ShapeDtypeStruct((k, D), jnp.float32),
                    jax.ShapeDtypeStruct((NW, 1, ET), jnp.int32),
                    jax.ShapeDtypeStruct((NW, 1, ET), jnp.int32))
        scratch = [
            pltpu.VMEM((n,), jnp.int32),
            pltpu.VMEM((ts,), jnp.int32),
            pltpu.VMEM((ET,), jnp.int32),
            pltpu.VMEM((ET,), jnp.int32),
            pltpu.VMEM((max(kw, klast), D), jnp.float32),
            pltpu.SemaphoreType.DMA,
        ]
    else:
        def body(perm_hbm, h_hbm, hg_hbm, pv, rows, sem):
            c = lax.axis_index("c")
            s = lax.axis_index("s")
            w = c * 16 + s
            pltpu.sync_copy(perm_hbm, pv)
            gather_rows(h_hbm, hg_hbm, pv, rows, sem, w)

        out_type = jax.ShapeDtypeStruct((k, D), jnp.float32)
        scratch = [
            pltpu.VMEM((n,), jnp.int32),
            pltpu.VMEM((max(kw, klast), D), jnp.float32),
            pltpu.SemaphoreType.DMA,
        ]

    return pl.kernel(
        body,
        out_type=out_type,
        mesh=plsc.VectorSubcoreMesh(**_MESH),
        compiler_params=pltpu.CompilerParams(needs_layout_passes=False),
        scratch_types=scratch,
    )


def kernel(x, adj, W_rel1, b_rel1, W_root1, W_rel2, b_rel2, W_root2, W_rel3,
           b_rel3, W_root3, wp1, wp2, wp3, lin1_W, lin1_b, lin2_W, lin2_b,
           lin3_W, lin3_b):
    f = x.shape[1]
    x_pad = jnp.pad(x, ((0, 0), (0, FP - f)))
    wr1 = jnp.pad(W_rel1, ((0, FP - f), (0, 0)))
    wo1 = jnp.pad(W_root1, ((0, FP - f), (0, 0)))
    row = lambda v: v.reshape(1, -1)

    src = adj[0].reshape(NW, 1, ET)
    dst = adj[1].reshape(NW, 1, ET)

    p1, r1 = _proj1(x_pad, wr1, wo1, row(b_rel1))
    part1 = _make_scatter(N0, N0)(p1, src, dst)
    h1, sc1, key1 = _act(part1, r1, row(wp1), N0)
    perm1, ssort1 = _make_sort(N0)(key1.reshape(N0), sc1.reshape(N0))
    hg1, sn1, dn1 = _make_select(N0, K1, True)(perm1, h1, src, dst)
    ss1 = ssort1[:K1].reshape(K1, 1)

    p2, r2, cm1, cs1 = _gproj(hg1, ss1, W_rel2, W_root2, row(b_rel2), K1)
    part2 = _make_scatter(K1, K1 + NDUM)(p2, sn1, dn1)
    h2, sc2, key2 = _act(part2[:, :K1, :], r2, row(wp2), K1)
    perm2, ssort2 = _make_sort(K1)(key2.reshape(K1), sc2.reshape(K1))
    hg2, sn2, dn2 = _make_select(K1, K2, True)(perm2, h2, sn1, dn1)
    ss2 = ssort2[:K2].reshape(K2, 1)

    p3, r3, cm2, cs2 = _gproj(hg2, ss2, W_rel3, W_root3, row(b_rel3), K2)
    part3 = _make_scatter(K2, K2 + NDUM)(p3, sn2, dn2)
    h3, sc3, key3 = _act(part3[:, :K2, :], r3, row(wp3), K2)
    perm3, ssort3 = _make_sort(K2)(key3.reshape(K2), sc3.reshape(K2))
    hg3 = _make_select(K2, K3, False)(perm3, h3)
    ss3 = ssort3[:K3].reshape(K3, 1)

    w3p = jnp.pad(lin3_W, ((0, 0), (0, D - 2)))
    b3p = jnp.pad(lin3_b, ((0, D - 2),))
    logits, prob, yhat = _head(hg3, ss3, cm1, cs1, cm2, cs2,
                               lin1_W, row(lin1_b), lin2_W, row(lin2_b),
                               w3p, row(b3p))
    return (logits, prob, yhat)


# speedup vs baseline: 15.7697x; 1.0823x over previous
import functools
import math

import jax
import jax.numpy as jnp
from jax import lax
from jax.experimental import pallas as pl
from jax.experimental.pallas import tpu as pltpu
from jax.experimental.pallas import tpu_sc as plsc

N0 = 10000
E = 320000
D = 128
FIN = 196
K1 = 8000
K2 = 6400
K3 = 5120
NDUM = 512
NW = 32
ET = E // NW
EW = 128

_MESH = dict(core_axis_name="c", subcore_axis_name="s", num_cores=2,
             num_subcores=16)


def _proj1_body(x_ref, wr_ref, wo_ref, b_ref, p_ref, r_ref):
    xx = x_ref[...]
    p_ref[...] = jnp.dot(xx, wr_ref[...], preferred_element_type=jnp.float32)
    r_ref[...] = jnp.dot(xx, wo_ref[...],
                         preferred_element_type=jnp.float32) + b_ref[...]


def _proj1(x_pad, wr, wo, b):
    bn = 400
    grid = (N0 // bn,)
    return pl.pallas_call(
        _proj1_body,
        grid=grid,
        in_specs=[
            pl.BlockSpec((bn, FIN), lambda i: (i, 0)),
            pl.BlockSpec((FIN, D), lambda i: (0, 0)),
            pl.BlockSpec((FIN, D), lambda i: (0, 0)),
            pl.BlockSpec((1, D), lambda i: (0, 0)),
        ],
        out_specs=[
            pl.BlockSpec((bn, D), lambda i: (i, 0)),
            pl.BlockSpec((bn, D), lambda i: (i, 0)),
        ],
        out_shape=[
            jax.ShapeDtypeStruct((N0, D), jnp.float32),
            jax.ShapeDtypeStruct((N0, D), jnp.float32),
        ],
    )(x_pad, wr, wo, b)


def _gproj_body(hg_ref, ss_ref, wr_ref, wo_ref, b_ref, p_ref, r_ref,
                cm_ref, cs_ref):
    i = pl.program_id(0)
    g = hg_ref[...] * ss_ref[...]
    p_ref[...] = jnp.dot(g, wr_ref[...], preferred_element_type=jnp.float32)
    r_ref[...] = jnp.dot(g, wo_ref[...],
                         preferred_element_type=jnp.float32) + b_ref[...]
    bmax = jnp.max(g, axis=0, keepdims=True)
    bsum = jnp.sum(g, axis=0, keepdims=True)

    @pl.when(i == 0)
    def _():
        cm_ref[...] = bmax
        cs_ref[...] = bsum

    @pl.when(i > 0)
    def _():
        cm_ref[...] = jnp.maximum(cm_ref[...], bmax)
        cs_ref[...] = cs_ref[...] + bsum


def _gproj(hg, ss, wr, wo, b, k):
    bn = 400
    grid = (k // bn,)
    return pl.pallas_call(
        _gproj_body,
        grid=grid,
        in_specs=[
            pl.BlockSpec((bn, D), lambda i: (i, 0)),
            pl.BlockSpec((bn, 1), lambda i: (i, 0)),
            pl.BlockSpec((D, D), lambda i: (0, 0)),
            pl.BlockSpec((D, D), lambda i: (0, 0)),
            pl.BlockSpec((1, D), lambda i: (0, 0)),
        ],
        out_specs=[
            pl.BlockSpec((bn, D), lambda i: (i, 0)),
            pl.BlockSpec((bn, D), lambda i: (i, 0)),
            pl.BlockSpec((1, D), lambda i: (0, 0)),
            pl.BlockSpec((1, D), lambda i: (0, 0)),
        ],
        out_shape=[
            jax.ShapeDtypeStruct((k, D), jnp.float32),
            jax.ShapeDtypeStruct((k, D), jnp.float32),
            jax.ShapeDtypeStruct((1, D), jnp.float32),
            jax.ShapeDtypeStruct((1, D), jnp.float32),
        ],
    )(hg, ss, wr, wo, b)


def _act_body(pa_ref, r_ref, w_ref, h_ref, sc_ref, key_ref):
    agg = pa_ref[0] + pa_ref[1] + r_ref[...]
    h = jnp.maximum(agg, 0.0)
    h_ref[...] = h
    w = w_ref[...]
    nrm = jnp.sqrt(jnp.sum(w * w))
    z = jnp.sum(h * w, axis=1, keepdims=True)
    sc = jnp.tanh(z / nrm)
    sc_ref[...] = sc
    u = lax.bitcast_convert_type(jnp.where(sc == 0.0, 0.0, sc), jnp.int32)
    kasc = jnp.where(u < 0, ~u, u ^ jnp.int32(-2147483648))
    key_ref[...] = ~kasc


def _act(partials, r, w_row, n):
    bn = 400
    grid = (n // bn,)
    return pl.pallas_call(
        _act_body,
        grid=grid,
        in_specs=[
            pl.BlockSpec((2, bn, D), lambda i: (0, i, 0)),
            pl.BlockSpec((bn, D), lambda i: (i, 0)),
            pl.BlockSpec((1, D), lambda i: (0, 0)),
        ],
        out_specs=[
            pl.BlockSpec((bn, D), lambda i: (i, 0)),
            pl.BlockSpec((bn, 1), lambda i: (i, 0)),
            pl.BlockSpec((bn, 1), lambda i: (i, 0)),
        ],
        out_shape=[
            jax.ShapeDtypeStruct((n, D), jnp.float32),
            jax.ShapeDtypeStruct((n, 1), jnp.float32),
            jax.ShapeDtypeStruct((n, 1), jnp.int32),
        ],
    )(partials, r, w_row)


def _head_body(hg_ref, ss_ref, cm1_ref, cs1_ref, cm2_ref, cs2_ref,
               w1_ref, b1_ref, w2_ref, b2_ref, w3_ref, b3_ref,
               lg_ref, pr_ref, yh_ref, cm_acc, cs_acc):
    i = pl.program_id(0)
    n = pl.num_programs(0)
    g = hg_ref[...] * ss_ref[...]
    bmax = jnp.max(g, axis=0, keepdims=True)
    bsum = jnp.sum(g, axis=0, keepdims=True)

    @pl.when(i == 0)
    def _():
        cm_acc[...] = bmax
        cs_acc[...] = bsum

    @pl.when(i > 0)
    def _():
        cm_acc[...] = jnp.maximum(cm_acc[...], bmax)
        cs_acc[...] = cs_acc[...] + bsum

    @pl.when(i == n - 1)
    def _():
        xmax = cm1_ref[...] + cm2_ref[...] + cm_acc[...]
        xmean = (cs1_ref[...] / K1 + cs2_ref[...] / K2 + cs_acc[...] / K3)
        xh = jnp.concatenate([xmax, xmean], axis=1)
        z1 = jnp.maximum(
            jnp.dot(xh, w1_ref[...], preferred_element_type=jnp.float32)
            + b1_ref[...], 0.0)
        z2 = jnp.maximum(
            jnp.dot(z1, w2_ref[...], preferred_element_type=jnp.float32)
            + b2_ref[...], 0.0)
        lfull = jnp.dot(z2, w3_ref[...], preferred_element_type=jnp.float32)
        logits = lfull[:, :2] + b3_ref[...][:, :2]
        lg_ref[...] = logits
        m = jnp.max(logits, axis=1, keepdims=True)
        e = jnp.exp(logits - m)
        pr_ref[...] = e / jnp.sum(e, axis=1, keepdims=True)
        yh_ref[...] = jnp.where(logits[:, 0:1] >= logits[:, 1:2], 0, 1
                                ).astype(jnp.int32)


def _head(hg3, ss3, cm1, cs1, cm2, cs2, w1, b1, w2, b2, w3p, b3p):
    bn = 512
    grid = (K3 // bn,)
    full = lambda shape: pl.BlockSpec(shape, lambda i: tuple(0 for _ in shape))
    return pl.pallas_call(
        _head_body,
        grid=grid,
        in_specs=[
            pl.BlockSpec((bn, D), lambda i: (i, 0)),
            pl.BlockSpec((bn, 1), lambda i: (i, 0)),
            full((1, D)), full((1, D)), full((1, D)), full((1, D)),
            full((2 * D, D)), full((1, D)),
            full((D, D // 2)), full((1, D // 2)),
            full((D // 2, D)), full((1, D)),
        ],
        out_specs=[
            full((1, 2)), full((1, 2)), full((1, 1)),
        ],
        out_shape=[
            jax.ShapeDtypeStruct((1, 2), jnp.float32),
            jax.ShapeDtypeStruct((1, 2), jnp.float32),
            jax.ShapeDtypeStruct((1, 1), jnp.int32),
        ],
        scratch_shapes=[
            pltpu.VMEM((1, D), jnp.float32),
            pltpu.VMEM((1, D), jnp.float32),
        ],
    )(hg3, ss3, cm1, cs1, cm2, cs2, w1, b1, w2, b2, w3p, b3p)


def _make_scatter(nsrc, nrows):
    chunk = -(-(nrows // 16) // 8) * 8
    last = nrows - 15 * chunk
    assert last > 0 and last % 8 == 0 and nrows % 8 == 0

    def body(p_hbm, s_hbm, d_hbm, out_hbm, sv_a, dv_a, sv_b, dv_b, sv_t,
             dv_t, rows_a, rows_b, acc, sem_a, sem_b):
        c = lax.axis_index("c")
        s = lax.axis_index("s")
        w = c * 16 + s

        def zrow(i, _):
            for j in range(D // 16):
                rows_a[i, pl.ds(j * 16, 16)] = jnp.zeros((16,), jnp.float32)
            return 0
        lax.fori_loop(0, EW, zrow, 0)

        def zero_and_out(cnt, do_out):
            step = EW
            for off in range(0, cnt, step):
                cc = min(step, cnt - off)
                if do_out:
                    pltpu.sync_copy(
                        acc.at[pl.ds(s * chunk + off, cc), :],
                        out_hbm.at[c, pl.ds(s * chunk + off, cc), :])
                else:
                    pltpu.sync_copy(rows_a.at[pl.ds(0, cc), :],
                                    acc.at[pl.ds(s * chunk + off, cc), :])

        @pl.when(s < 15)
        def _():
            zero_and_out(chunk, False)

        @pl.when(s == 15)
        def _():
            zero_and_out(last, False)
        plsc.subcore_barrier()

        nf = ET // EW

        def stage(i, sref, dref):
            pltpu.sync_copy(s_hbm.at[w, 0, pl.ds(i * EW, EW)], sref)
            pltpu.sync_copy(d_hbm.at[w, 0, pl.ds(i * EW, EW)], dref)

        def gat(sref, rbuf, sm):
            pltpu.async_copy(p_hbm.at[sref], rbuf, sm)

        def wad(sref, dref, rbuf, sm):
            pltpu.make_async_copy(p_hbm.at[sref], rbuf, sm).wait()
            pltpu.sync_copy(rbuf, acc.at[dref], add=True)

        stage(0, sv_a, dv_a)
        gat(sv_a, rows_a, sem_a)

        def pair(i2, _):
            a = 2 * i2
            stage(a + 1, sv_b, dv_b)
            gat(sv_b, rows_b, sem_b)
            wad(sv_a, dv_a, rows_a, sem_a)

            @pl.when(a + 2 < nf)
            def _():
                stage(a + 2, sv_a, dv_a)
                gat(sv_a, rows_a, sem_a)
            wad(sv_b, dv_b, rows_b, sem_b)
            return 0
        lax.fori_loop(0, nf // 2, pair, 0)
        if nf % 2:
            wad(sv_a, dv_a, rows_a, sem_a)
        if ET % EW:
            tl = ET % EW
            tb = ET - tl
            pltpu.sync_copy(s_hbm.at[w, 0, pl.ds(tb, tl)], sv_t)
            pltpu.sync_copy(d_hbm.at[w, 0, pl.ds(tb, tl)], dv_t)
            pltpu.async_copy(p_hbm.at[sv_t], rows_a.at[pl.ds(0, tl), :],
                             sem_a).wait()
            pltpu.sync_copy(rows_a.at[pl.ds(0, tl), :], acc.at[dv_t],
                            add=True)
        plsc.subcore_barrier()

        @pl.when(s < 15)
        def _():
            zero_and_out(chunk, True)

        @pl.when(s == 15)
        def _():
            zero_and_out(last, True)

    return pl.kernel(
        body,
        out_type=jax.ShapeDtypeStruct((2, nrows, D), jnp.float32),
        mesh=plsc.VectorSubcoreMesh(**_MESH),
        compiler_params=pltpu.CompilerParams(needs_layout_passes=False),
        scratch_types=[
            pltpu.VMEM((EW,), jnp.int32),
            pltpu.VMEM((EW,), jnp.int32),
            pltpu.VMEM((EW,), jnp.int32),
            pltpu.VMEM((EW,), jnp.int32),
            pltpu.VMEM((max(ET % EW, 8),), jnp.int32),
            pltpu.VMEM((max(ET % EW, 8),), jnp.int32),
            pltpu.VMEM((EW, D), jnp.float32),
            pltpu.VMEM((EW, D), jnp.float32),
            pltpu.VMEM_SHARED((nrows, D), jnp.float32),
            pltpu.SemaphoreType.DMA,
            pltpu.SemaphoreType.DMA,
        ],
    )


def _make_sort(n):
    nv = n // 16

    def body(key_hbm, score_hbm, perm_hbm, ssort_hbm, sv, fb, ka, va, kb, vb,
             hist, offs):
        c = lax.axis_index("c")
        s = lax.axis_index("s")

        @pl.when(jnp.logical_and(c == 0, s == 0))
        def _():
            pltpu.sync_copy(key_hbm, ka)
            pltpu.sync_copy(score_hbm, sv)
            iota = lax.iota(jnp.int32, 16)

            def xform(i, _):
                va[pl.ds(i * 16, 16)] = iota + i * 16
                return 0
            lax.fori_loop(0, nv, xform, 0)

            for p in range(3):
                sk, svals, dk, dvals = ((ka, va, kb, vb) if p % 2 == 0
                                        else (kb, vb, ka, va))
                sh = jnp.int32(11 * p)
                m255 = jnp.int32(2047)

                def zh(j, _):
                    hist[pl.ds(j * 16, 16)] = jnp.zeros((16,), jnp.int32)
                    return 0
                lax.fori_loop(0, 128, zh, 0)

                def hloop(i, _):
                    kv = sk[pl.ds(i * 16, 16)]
                    d = lax.shift_right_logical(kv, sh) & m255
                    occ, last = plsc.scan_count(d)
                    plsc.addupdate_scatter(
                        hist, [d], occ.astype(jnp.int32), mask=last)
                    return 0
                lax.fori_loop(0, nv, hloop, 0)

                def pref(j, carry):
                    v = hist[pl.ds(j * 16, 16)]
                    inc = plsc.cumsum(v)
                    offs[pl.ds(j * 16, 16)] = inc - v + carry
                    return carry + jnp.sum(v)
                lax.fori_loop(0, 128, pref, jnp.int32(0))

                def sloop(i, _):
                    kv = sk[pl.ds(i * 16, 16)]
                    vv = svals[pl.ds(i * 16, 16)]
                    d = lax.shift_right_logical(kv, sh) & m255
                    occ, last = plsc.scan_count(d)
                    base = plsc.load_gather(offs, [d])
                    pos = base + occ.astype(jnp.int32) - 1
                    plsc.store_scatter(dk, [pos], kv)
                    plsc.store_scatter(dvals, [pos], vv)
                    plsc.addupdate_scatter(
                        offs, [d], occ.astype(jnp.int32), mask=last)
                    return 0
                lax.fori_loop(0, nv, sloop, 0)

            def unx(i, _):
                pvv = vb[pl.ds(i * 16, 16)]
                fb[pl.ds(i * 16, 16)] = plsc.load_gather(sv, [pvv])
                return 0
            lax.fori_loop(0, nv, unx, 0)

            pltpu.sync_copy(vb, perm_hbm)
            pltpu.sync_copy(fb, ssort_hbm)

    return pl.kernel(
        body,
        out_type=(jax.ShapeDtypeStruct((n,), jnp.int32),
                  jax.ShapeDtypeStruct((n,), jnp.float32)),
        mesh=plsc.VectorSubcoreMesh(**_MESH),
        compiler_params=pltpu.CompilerParams(needs_layout_passes=False),
        scratch_types=[
            pltpu.VMEM((n,), jnp.float32),
            pltpu.VMEM((n,), jnp.float32),
            pltpu.VMEM((n,), jnp.int32),
            pltpu.VMEM((n,), jnp.int32),
            pltpu.VMEM((n,), jnp.int32),
            pltpu.VMEM((n,), jnp.int32),
            pltpu.VMEM((2048,), jnp.int32),
            pltpu.VMEM((2048,), jnp.int32),
        ],
    )


def _make_select(n, k, relabel):
    ts = n + NDUM
    kw = (k // NW) & ~7
    klast = k - (NW - 1) * kw
    assert klast > 0 and klast % 8 == 0

    def build_table(perm_hbm, pv, table):
        pltpu.sync_copy(perm_hbm, pv)
        neg1 = jnp.full((16,), -1, jnp.int32)

        def init(i, _):
            table[pl.ds(i * 16, 16)] = neg1
            return 0
        lax.fori_loop(0, ts // 16, init, 0)
        iota = lax.iota(jnp.int32, 16)

        def rank(i, _):
            pvv = pv[pl.ds(i * 16, 16)]
            plsc.store_scatter(table, [pvv], iota + i * 16)
            return 0
        lax.fori_loop(0, k // 16, rank, 0)

    def gather_rows(h_hbm, hg_hbm, pv, rows, sem, w):
        rb = w * kw

        @pl.when(w < NW - 1)
        def _():
            pltpu.async_copy(h_hbm.at[pv.at[pl.ds(rb, kw)]],
                             rows.at[pl.ds(0, kw), :], sem).wait()
            pltpu.sync_copy(rows.at[pl.ds(0, kw), :],
                            hg_hbm.at[pl.ds(rb, kw), :])

        @pl.when(w == NW - 1)
        def _():
            pltpu.async_copy(h_hbm.at[pv.at[pl.ds(rb, klast)]],
                             rows.at[pl.ds(0, klast), :], sem).wait()
            pltpu.sync_copy(rows.at[pl.ds(0, klast), :],
                            hg_hbm.at[pl.ds(rb, klast), :])

    if relabel:
        def body(perm_hbm, h_hbm, so_hbm, do_hbm, hg_hbm, sn_hbm, dn_hbm,
                 pv, table, ev_s, ev_d, rows, sem):
            c = lax.axis_index("c")
            s = lax.axis_index("s")
            w = c * 16 + s
            build_table(perm_hbm, pv, table)
            pltpu.sync_copy(so_hbm.at[w, 0], ev_s)
            pltpu.sync_copy(do_hbm.at[w, 0], ev_d)
            iota = lax.iota(jnp.int32, 16)
            base = w * ET

            def rel(i, _):
                so = ev_s[pl.ds(i * 16, 16)]
                do = ev_d[pl.ds(i * 16, 16)]
                sn = plsc.load_gather(table, [so])
                dn = plsc.load_gather(table, [do])
                inval = jnp.logical_or(sn < 0, dn < 0)
                eidx = iota + (base + i * 16)
                ev_s[pl.ds(i * 16, 16)] = jnp.where(inval, eidx & 1023, sn)
                ev_d[pl.ds(i * 16, 16)] = jnp.where(inval, k + (eidx & 511),
                                                    dn)
                return 0
            lax.fori_loop(0, ET // 16, rel, 0)
            pltpu.sync_copy(ev_s, sn_hbm.at[w, 0])
            pltpu.sync_copy(ev_d, dn_hbm.at[w, 0])
            gather_rows(h_hbm, hg_hbm, pv, rows, sem, w)

        out_type = (jax.ShapeDtypeStruct((k, D), jnp.float32),
                    jax.ShapeDtypeStruct((NW, 1, ET), jnp.int32),
                    jax.ShapeDtypeStruct((NW, 1, ET), jnp.int32))
        scratch = [
            pltpu.VMEM((n,), jnp.int32),
            pltpu.VMEM((ts,), jnp.int32),
            pltpu.VMEM((ET,), jnp.int32),
            pltpu.VMEM((ET,), jnp.int32),
            pltpu.VMEM((max(kw, klast), D), jnp.float32),
            pltpu.SemaphoreType.DMA,
        ]
    else:
        def body(perm_hbm, h_hbm, hg_hbm, pv, rows, sem):
            c = lax.axis_index("c")
            s = lax.axis_index("s")
            w = c * 16 + s
            pltpu.sync_copy(perm_hbm, pv)
            gather_rows(h_hbm, hg_hbm, pv, rows, sem, w)

        out_type = jax.ShapeDtypeStruct((k, D), jnp.float32)
        scratch = [
            pltpu.VMEM((n,), jnp.int32),
            pltpu.VMEM((max(kw, klast), D), jnp.float32),
            pltpu.SemaphoreType.DMA,
        ]

    return pl.kernel(
        body,
        out_type=out_type,
        mesh=plsc.VectorSubcoreMesh(**_MESH),
        compiler_params=pltpu.CompilerParams(needs_layout_passes=False),
        scratch_types=scratch,
    )


def kernel(x, adj, W_rel1, b_rel1, W_root1, W_rel2, b_rel2, W_root2, W_rel3,
           b_rel3, W_root3, wp1, wp2, wp3, lin1_W, lin1_b, lin2_W, lin2_b,
           lin3_W, lin3_b):
    row = lambda v: v.reshape(1, -1)

    src = adj[0].reshape(NW, 1, ET)
    dst = adj[1].reshape(NW, 1, ET)

    p1, r1 = _proj1(x, W_rel1, W_root1, row(b_rel1))
    part1 = _make_scatter(N0, N0)(p1, src, dst)
    h1, sc1, key1 = _act(part1, r1, row(wp1), N0)
    perm1, ssort1 = _make_sort(N0)(key1.reshape(N0), sc1.reshape(N0))
    hg1, sn1, dn1 = _make_select(N0, K1, True)(perm1, h1, src, dst)
    ss1 = ssort1[:K1].reshape(K1, 1)

    p2, r2, cm1, cs1 = _gproj(hg1, ss1, W_rel2, W_root2, row(b_rel2), K1)
    part2 = _make_scatter(K1, K1 + NDUM)(p2, sn1, dn1)
    h2, sc2, key2 = _act(part2[:, :K1, :], r2, row(wp2), K1)
    perm2, ssort2 = _make_sort(K1)(key2.reshape(K1), sc2.reshape(K1))
    hg2, sn2, dn2 = _make_select(K1, K2, True)(perm2, h2, sn1, dn1)
    ss2 = ssort2[:K2].reshape(K2, 1)

    p3, r3, cm2, cs2 = _gproj(hg2, ss2, W_rel3, W_root3, row(b_rel3), K2)
    part3 = _make_scatter(K2, K2 + NDUM)(p3, sn2, dn2)
    h3, sc3, key3 = _act(part3[:, :K2, :], r3, row(wp3), K2)
    perm3, ssort3 = _make_sort(K2)(key3.reshape(K2), sc3.reshape(K2))
    hg3 = _make_select(K2, K3, False)(perm3, h3)
    ss3 = ssort3[:K3].reshape(K3, 1)

    w3p = jnp.pad(lin3_W, ((0, 0), (0, D - 2)))
    b3p = jnp.pad(lin3_b, ((0, D - 2),))
    logits, prob, yhat = _head(hg3, ss3, cm1, cs1, cm2, cs2,
                               lin1_W, row(lin1_b), lin2_W, row(lin2_b),
                               w3p, row(b3p))
    return (logits, prob, yhat)
